# Initial kernel scaffold; baseline (speedup 1.0000x reference)
import jax
import jax.numpy as jnp
from jax.experimental import pallas as pl


def kernel(boxes, scores):
    raise NotImplementedError("write your pallas kernel here")


# speedup vs baseline: 3.8840x; 3.8840x over previous
import jax
import jax.numpy as jnp
from jax.experimental import pallas as pl

PRE = 2000
POST = 1000
THR = 0.7
NEG = float("-inf")
CHUNK = 2048


def _partner(x, j):
    n = x.shape[-1]
    left = jnp.concatenate([x[..., j:], x[..., :j]], axis=-1)
    right = jnp.concatenate([x[..., n - j:], x[..., :n - j]], axis=-1)
    bit = (jax.lax.broadcasted_iota(jnp.int32, (1, 1, n), 2) & j) != 0
    return jnp.where(bit, right, left)


def _ce(arrs, j, want_big):
    s, ii = arrs[0], arrs[1]
    ps = _partner(s, j)
    pi = _partner(ii, j)
    self_first = (s > ps) | ((s == ps) & (ii < pi))
    take = want_big ^ self_first
    out = [jnp.where(take, ps, s), jnp.where(take, pi, ii)]
    for a in arrs[2:]:
        out.append(jnp.where(take, _partner(a, j), a))
    return out


def _sort_desc(arrs):
    n = arrs[0].shape[-1]
    iota = jax.lax.broadcasted_iota(jnp.int32, (1, 1, n), 2)
    k = 2
    while k <= n:
        j = k // 2
        while j >= 1:
            dir_desc = (iota & k) == 0
            lower = (iota & j) == 0
            arrs = _ce(arrs, j, dir_desc == lower)
            j //= 2
        k *= 2
    return arrs


def _merge_desc(arrs):
    n = arrs[0].shape[-1]
    iota = jax.lax.broadcasted_iota(jnp.int32, (1, 1, n), 2)
    j = n // 2
    while j >= 1:
        arrs = _ce(arrs, j, (iota & j) == 0)
        j //= 2
    return arrs


def _flip(x):
    n = x.shape[-1]
    j = 1
    while j < n:
        x = _partner(x, j)
        j *= 2
    return x


def _combine(A, B):
    Br = [_flip(b) for b in B]
    a_first = (A[0] > Br[0]) | ((A[0] == Br[0]) & (A[1] < Br[1]))
    C = [jnp.where(a_first, a, br) for a, br in zip(A, Br)]
    return _merge_desc(C)


def _topk_reduce(arrs):
    while arrs[0].shape[1] > 1:
        c = arrs[0].shape[1]
        h = c // 2
        A = [a[:, :h] for a in arrs]
        Bp = [a[:, h:2 * h] for a in arrs]
        M = _combine(A, Bp)
        if c % 2:
            arrs = [jnp.concatenate([m, a[:, 2 * h:]], axis=1)
                    for m, a in zip(M, arrs)]
        else:
            arrs = M
    return [a[:, 0] for a in arrs]


def _body(s_ref, i_ref, b0_ref, b1_ref, b2_ref, b3_ref, out_ref):
    arrs = [s_ref[...], i_ref[...], b0_ref[...], b1_ref[...],
            b2_ref[...], b3_ref[...]]
    arrs = _sort_desc(arrs)
    score, _, o0, o1, o2, o3 = _topk_reduce(arrs)

    B, n = score.shape
    y1 = jnp.minimum(o0, o2)
    y2 = jnp.maximum(o0, o2)
    x1 = jnp.minimum(o1, o3)
    x2 = jnp.maximum(o1, o3)
    areas = (y2 - y1) * (x2 - x1)
    lane = jax.lax.broadcasted_iota(jnp.int32, (B, n), 1)
    work0 = jnp.where(lane < PRE, score, NEG)

    def body(i, work):
        alive = work > NEG
        idxsel = jnp.min(jnp.where(alive, lane, 2 * n), axis=1, keepdims=True)
        valid = idxsel < 2 * n
        onehot = lane == idxsel

        def ext(a):
            return jnp.sum(jnp.where(onehot, a, 0.0), axis=1, keepdims=True)

        s0 = ext(o0)
        s1 = ext(o1)
        s2 = ext(o2)
        s3 = ext(o3)
        sc = ext(score)
        by1 = jnp.minimum(s0, s2)
        by2 = jnp.maximum(s0, s2)
        bx1 = jnp.minimum(s1, s3)
        bx2 = jnp.maximum(s1, s3)
        barea = (by2 - by1) * (bx2 - bx1)
        ih = jnp.maximum(jnp.minimum(by2, y2) - jnp.maximum(by1, y1), 0.0)
        iw = jnp.maximum(jnp.minimum(bx2, x2) - jnp.maximum(bx1, x1), 0.0)
        inter = ih * iw
        union = barea + areas - inter
        iou = jnp.where(union > 0, inter / union, 0.0)
        new_work = jnp.where(iou > THR, NEG, work)

        zero = jnp.zeros_like(sc)
        o0c = jnp.where(valid, jnp.clip(s0, 0.0, 1.0), zero)
        o1c = jnp.where(valid, jnp.clip(s1, 0.0, 1.0), zero)
        o2c = jnp.where(valid, jnp.clip(s2, 0.0, 1.0), zero)
        o3c = jnp.where(valid, jnp.clip(s3, 0.0, 1.0), zero)
        scc = jnp.where(valid, sc, zero)
        vals = jnp.concatenate([o0c, o1c, o2c, o3c, scc], axis=1)
        out_ref[pl.ds(i, 1), :, :] = vals.reshape(1, B, 5)
        return new_work

    jax.lax.fori_loop(0, POST, body, work0)


def kernel(boxes, scores):
    boxes = boxes.astype(jnp.float32)
    s = jnp.squeeze(scores.astype(jnp.float32), -1)
    B, N = s.shape
    nch = -(-N // CHUNK)
    Np = nch * CHUNK
    pad = Np - N

    s_p = jnp.pad(s, ((0, 0), (0, pad)), constant_values=NEG)
    s_p = s_p.reshape(B, nch, CHUNK)
    idx = jnp.broadcast_to(jnp.arange(Np, dtype=jnp.int32)[None, :], (B, Np))
    idx = idx.reshape(B, nch, CHUNK)
    bx = jnp.pad(boxes, ((0, 0), (0, pad), (0, 0)))
    coords = [bx[..., j].reshape(B, nch, CHUNK) for j in range(4)]

    out = pl.pallas_call(
        _body,
        out_shape=jax.ShapeDtypeStruct((1024, B, 5), jnp.float32),
    )(s_p, idx, *coords)

    nb = jnp.transpose(out[:POST, :, :4], (1, 0, 2))
    ns = jnp.transpose(out[:POST, :, 4], (1, 0))
    return nb, ns
